# Initial kernel scaffold; baseline (speedup 1.0000x reference)
import jax
import jax.numpy as jnp
from jax.experimental import pallas as pl


def kernel(x, keys, prompts, layer):
    raise NotImplementedError("write your pallas kernel here")


# speedup vs baseline: 2.7800x; 2.7800x over previous
import functools

import jax
import jax.numpy as jnp
from jax import lax
from jax.experimental import pallas as pl
from jax.experimental.pallas import tpu as pltpu
from jax.experimental.pallas import tpu_sc as plsc

_TOPK = 9
_POOL = 20
_DIM = 768
_B = 16384
_BLK = 2048
_NW = 32
_RPW = _B // _NW
_LANES = 16


def _batcher_pairs(n):
    pairs = []
    t = (n - 1).bit_length()
    p = 1 << (t - 1)
    while p > 0:
        q = 1 << (t - 1)
        r = 0
        d = p
        while True:
            for i in range(n - d):
                if (i & p) == r:
                    pairs.append((i, i + d))
            if q == p:
                break
            d = q - p
            q >>= 1
            r = p
        p >>= 1
    return pairs


_PAIRS = _batcher_pairs(_POOL)


def _tc_body(x_ref, kn_ref, out_ref):
    x = x_ref[...]
    kn = kn_ref[...]
    kn_ssq = jnp.sum(kn * kn, axis=1, keepdims=True)
    kn_n = kn / jnp.maximum(jnp.sqrt(kn_ssq), 1e-12)
    ones = jnp.ones((1, _DIM), jnp.float32)
    ssq = lax.dot_general(ones, x * x, (((1,), (1,)), ((), ())),
                          preferred_element_type=jnp.float32)
    inv = 1.0 / jnp.maximum(jnp.sqrt(ssq), 1e-12)
    simt = lax.dot_general(kn_n, x, (((1,), (1,)), ((), ())),
                           preferred_element_type=jnp.float32)
    out_ref[...] = simt * inv


def _tc_stage(x, keys_l):
    return pl.pallas_call(
        _tc_body,
        grid=(_B // _BLK,),
        in_specs=[
            pl.BlockSpec((_BLK, _DIM), lambda i: (i, 0)),
            pl.BlockSpec((_POOL, _DIM), lambda i: (0, 0)),
        ],
        out_specs=pl.BlockSpec((_POOL, _BLK), lambda i: (0, i)),
        out_shape=jax.ShapeDtypeStruct((_POOL, _B), jnp.float32),
    )(x, keys_l)


def _sc_stage(simt):
    mesh = plsc.VectorSubcoreMesh(core_axis_name="c", subcore_axis_name="s")

    @functools.partial(
        pl.kernel,
        mesh=mesh,
        out_type=jax.ShapeDtypeStruct((_NW * _LANES,), jnp.float32),
        scratch_types=[
            pltpu.VMEM((_POOL, _RPW), jnp.float32),
            pltpu.VMEM((_LANES,), jnp.float32),
        ],
    )
    def k(simt_hbm, out_hbm, sim_v, acc_v):
        wid = lax.axis_index("s") * 2 + lax.axis_index("c")
        base = wid * _RPW
        pltpu.sync_copy(simt_hbm.at[:, pl.ds(base, _RPW)], sim_v)

        def body(g, acc):
            start = pl.multiple_of(g * _LANES, _LANES)
            vs = [sim_v[j, pl.ds(start, _LANES)] for j in range(_POOL)]
            for (i, j) in _PAIRS:
                lo = jnp.minimum(vs[i], vs[j])
                hi = jnp.maximum(vs[i], vs[j])
                vs[i] = lo
                vs[j] = hi
            s = vs[_POOL - _TOPK]
            for t in range(_POOL - _TOPK + 1, _POOL):
                s = s + vs[t]
            return acc + s

        acc = lax.fori_loop(0, _RPW // _LANES, body,
                            jnp.zeros((_LANES,), jnp.float32))
        acc_v[...] = acc
        pltpu.sync_copy(acc_v, out_hbm.at[pl.ds(wid * _LANES, _LANES)])

    return k(simt)


def kernel(x, keys, prompts, layer):
    keys_l = lax.dynamic_index_in_dim(keys, layer, 0, keepdims=False)
    simt = _tc_stage(x, keys_l)
    partials = _sc_stage(simt)
    total = jnp.sum(partials)
    return (jnp.float32(1.0) - total / jnp.float32(_B * _TOPK)).astype(jnp.float32)


# speedup vs baseline: 2.8321x; 1.0188x over previous
import functools

import jax
import jax.numpy as jnp
from jax import lax
from jax.experimental import pallas as pl
from jax.experimental.pallas import tpu as pltpu
from jax.experimental.pallas import tpu_sc as plsc

_TOPK = 9
_POOL = 20
_DIM = 768
_B = 16384
_BLK = 4096
_NW = 32
_RPW = _B // _NW
_LANES = 16


def _batcher_pairs(n):
    pairs = []
    t = (n - 1).bit_length()
    p = 1 << (t - 1)
    while p > 0:
        q = 1 << (t - 1)
        r = 0
        d = p
        while True:
            for i in range(n - d):
                if (i & p) == r:
                    pairs.append((i, i + d))
            if q == p:
                break
            d = q - p
            q >>= 1
            r = p
        p >>= 1
    return pairs


_PAIRS = _batcher_pairs(_POOL)


def _tc_body(x_ref, kn_ref, out_ref):
    x = x_ref[...]
    kn = kn_ref[...]
    kn_ssq = jnp.sum(kn * kn, axis=1, keepdims=True)
    kn_n = kn / jnp.maximum(jnp.sqrt(kn_ssq), 1e-12)
    ones = jnp.ones((1, _DIM), jnp.float32)
    ssq = lax.dot_general(ones, x * x, (((1,), (1,)), ((), ())),
                          preferred_element_type=jnp.float32)
    inv = 1.0 / jnp.maximum(jnp.sqrt(ssq), 1e-12)
    simt = lax.dot_general(kn_n, x, (((1,), (1,)), ((), ())),
                           preferred_element_type=jnp.float32)
    out_ref[...] = simt * inv


def _tc_stage(x, keys_l):
    return pl.pallas_call(
        _tc_body,
        grid=(_B // _BLK,),
        in_specs=[
            pl.BlockSpec((_BLK, _DIM), lambda i: (i, 0)),
            pl.BlockSpec((_POOL, _DIM), lambda i: (0, 0)),
        ],
        out_specs=pl.BlockSpec((_POOL, _BLK), lambda i: (0, i)),
        out_shape=jax.ShapeDtypeStruct((_POOL, _B), jnp.float32),
    )(x, keys_l)


def _sc_stage(simt):
    mesh = plsc.VectorSubcoreMesh(core_axis_name="c", subcore_axis_name="s")

    @functools.partial(
        pl.kernel,
        mesh=mesh,
        out_type=jax.ShapeDtypeStruct((_NW * _LANES,), jnp.float32),
        scratch_types=[
            pltpu.VMEM((_POOL, _RPW), jnp.float32),
            pltpu.VMEM((_LANES,), jnp.float32),
        ],
    )
    def k(simt_hbm, out_hbm, sim_v, acc_v):
        wid = lax.axis_index("s") * 2 + lax.axis_index("c")
        base = wid * _RPW
        pltpu.sync_copy(simt_hbm.at[:, pl.ds(base, _RPW)], sim_v)

        def body(g, acc):
            start = pl.multiple_of(g * _LANES, _LANES)
            vs = [sim_v[j, pl.ds(start, _LANES)] for j in range(_POOL)]
            for (i, j) in _PAIRS:
                lo = jnp.minimum(vs[i], vs[j])
                hi = jnp.maximum(vs[i], vs[j])
                vs[i] = lo
                vs[j] = hi
            s = vs[_POOL - _TOPK]
            for t in range(_POOL - _TOPK + 1, _POOL):
                s = s + vs[t]
            return acc + s

        acc = lax.fori_loop(0, _RPW // _LANES, body,
                            jnp.zeros((_LANES,), jnp.float32))
        acc_v[...] = acc
        pltpu.sync_copy(acc_v, out_hbm.at[pl.ds(wid * _LANES, _LANES)])

    return k(simt)


def kernel(x, keys, prompts, layer):
    keys_l = lax.dynamic_index_in_dim(keys, layer, 0, keepdims=False)
    simt = _tc_stage(x, keys_l)
    partials = _sc_stage(simt)
    total = jnp.sum(partials)
    return (jnp.float32(1.0) - total / jnp.float32(_B * _TOPK)).astype(jnp.float32)
